# baseline (speedup 1.0000x reference)
import functools

import jax
import jax.numpy as jnp
from jax import lax
from jax.experimental import pallas as pl
from jax.experimental.pallas import tpu as pltpu
from jax.experimental.pallas import tpu_sc as plsc

NUM_USERS = 1000000
NUM_ITEMS = 1000000
EMB = 32
HID = 64
B = 16384

NC = 2
NS = 16
NW = NC * NS
LANES = 128
TCOLS = (NUM_USERS + LANES - 1) // LANES
SPW = -(-TCOLS // NW)
WT = 4
WAVE = WT * LANES
NWAVES = -(-SPW // WT)
STRIPE = SPW * LANES
SLACK = 768
NGRP = SLACK // 16
WSLACK = 64
OUTROWS = B + 16


def _scan_idx(ids_v, lo, hi, hit_r, hit_p):
    lanei = lax.iota(jnp.int32, 16)

    def body(v, off):
        rvec = ids_v[pl.ds(v * 16, 16)]
        pvec = lanei + v * 16
        m = (rvec >= lo) & (rvec < hi)
        o = off[0]
        plsc.store_compressed(hit_r.at[pl.ds(o, 16)], rvec, mask=m)
        plsc.store_compressed(hit_p.at[pl.ds(o, 16)], pvec, mask=m)
        return off + plsc.all_reduce_population_count(m)

    off = lax.fori_loop(0, B // 16, body, jnp.zeros((16,), jnp.int32))
    return off[0]


def _gather_body(uid_hbm, iid_hbm, utabT_hbm, itabT_hbm, uout_hbm, iout_hbm,
                 uids_v, iids_v, uhr, uhp, ihr, ihp, ubufs, ibufs,
                 uwr, uwp, iwr, iwp, batch_v, bpos_v, fsems, ssem):
    c = lax.axis_index("c")
    s = lax.axis_index("s")
    wid = s * NC + c
    lo = wid * STRIPE
    hi = lo + STRIPE

    rows0 = lax.iota(jnp.int32, 16)
    rows1 = rows0 + 16
    lanei = rows0

    def wave_off(k):
        col = jnp.minimum(wid * SPW + k * WT, TCOLS - WT)
        return pl.multiple_of(col * LANES, LANES)

    def fetch(k, slot):
        off = wave_off(k)
        pltpu.async_copy(utabT_hbm.at[:, pl.ds(off, WAVE)], ubufs[slot],
                         fsems[2 * slot])
        pltpu.async_copy(itabT_hbm.at[:, pl.ds(off, WAVE)], ibufs[slot],
                         fsems[2 * slot + 1])

    def wait_fetch(slot):
        pltpu.make_async_copy(utabT_hbm.at[:, pl.ds(0, WAVE)], ubufs[slot],
                              fsems[2 * slot]).wait()
        pltpu.make_async_copy(itabT_hbm.at[:, pl.ds(0, WAVE)], ibufs[slot],
                              fsems[2 * slot + 1]).wait()

    fetch(0, 0)
    fetch(1, 1)

    pltpu.sync_copy(uid_hbm, uids_v)
    pltpu.sync_copy(iid_hbm, iids_v)
    ucnt = _scan_idx(uids_v, lo, hi, uhr, uhp)
    icnt = _scan_idx(iids_v, lo, hi, ihr, ihp)

    def extract_wave(k, slot, buf, hr, hp, cnt, wr, wp, out_hbm):
        wlo = lo + k * WAVE
        whi = wlo + WAVE
        fbase = wave_off(k)

        def scan(v, off):
            rvec = hr[pl.ds(v * 16, 16)]
            pvec = hp[pl.ds(v * 16, 16)]
            valid = (lanei + v * 16) < cnt
            m = (rvec >= wlo) & (rvec < whi) & valid
            o = off[0]
            plsc.store_compressed(wr.at[pl.ds(o, 16)], rvec, mask=m)
            plsc.store_compressed(wp.at[pl.ds(o, 16)], pvec, mask=m)
            return off + plsc.all_reduce_population_count(m)

        wcnt = lax.fori_loop(0, NGRP, scan, jnp.zeros((16,), jnp.int32))[0]

        def group(g, carry):
            rvec = wr[pl.ds(g * 16, 16)]
            pvec = wp[pl.ds(g * 16, 16)]
            rem = wcnt - g * 16
            pm = jnp.where(lanei < rem, pvec, B + wid % 16)
            bpos_v[pl.ds(0, 16)] = pm
            for j in range(16):
                @pl.when(j < rem)
                def _():
                    colv = jnp.full((16,), rvec[j] - fbase, jnp.int32)
                    batch_v[j, pl.ds(0, 16)] = plsc.load_gather(
                        buf, [rows0, colv])
                    batch_v[j, pl.ds(16, 16)] = plsc.load_gather(
                        buf, [rows1, colv])
            pltpu.async_copy(batch_v, out_hbm.at[bpos_v], ssem).wait()
            return carry

        lax.fori_loop(0, (wcnt + 15) // 16, group, 0)

    def wave_pair(g, carry):
        for slot in range(2):
            k = g * 2 + slot
            wait_fetch(slot)
            extract_wave(k, slot, ubufs[slot], uhr, uhp, ucnt, uwr, uwp,
                         uout_hbm)
            extract_wave(k, slot, ibufs[slot], ihr, ihp, icnt, iwr, iwp,
                         iout_hbm)
            nxt = k + 2

            @pl.when(nxt < NWAVES)
            def _():
                fetch(nxt, slot)
        return carry

    lax.fori_loop(0, NWAVES // 2, wave_pair, 0)


def _gather_entry(uid_hbm, iid_hbm, utabT_hbm, itabT_hbm, uout, iout, *scr):
    (uids_v, iids_v, uhr, uhp, ihr, ihp, ub0, ub1, ib0, ib1,
     uwr, uwp, iwr, iwp, batch_v, bpos_v, f0, f1, f2, f3, ssem) = scr
    _gather_body(uid_hbm, iid_hbm, utabT_hbm, itabT_hbm, uout, iout,
                 uids_v, iids_v, uhr, uhp, ihr, ihp, [ub0, ub1], [ib0, ib1],
                 uwr, uwp, iwr, iwp, batch_v, bpos_v, [f0, f1, f2, f3], ssem)


@functools.cache
def _sc_gather_fn():
    wavebuf = pltpu.VMEM((EMB, WAVE), jnp.float32)
    hits = pltpu.VMEM((SLACK,), jnp.int32)
    whits = pltpu.VMEM((WSLACK,), jnp.int32)
    out_t = jax.ShapeDtypeStruct((OUTROWS, LANES), jnp.float32)
    return pl.kernel(
        _gather_entry,
        out_type=[out_t, out_t],
        mesh=plsc.VectorSubcoreMesh(core_axis_name="c", subcore_axis_name="s"),
        scratch_types=(
            [pltpu.VMEM((B,), jnp.int32)] * 2
            + [hits] * 4
            + [wavebuf] * 4
            + [whits] * 4
            + [pltpu.VMEM((16, LANES), jnp.float32),
               pltpu.VMEM((16,), jnp.int32)]
            + [pltpu.SemaphoreType.DMA] * 5
        ),
        compiler_params=pltpu.CompilerParams(use_tc_tiling_on_sc=True,
                                             needs_layout_passes=False),
    )


MLP_BLOCK = 2048


def _mlp_body(ue_ref, ie_ref, w1u_ref, w1i_ref, b1_ref,
              w2_ref, b2_ref, w3_ref, b3_ref, out_ref):
    ue = ue_ref[...][:, :EMB]
    ie = ie_ref[...][:, :EMB]
    h = (jnp.dot(ue, w1u_ref[...], preferred_element_type=jnp.float32)
         + jnp.dot(ie, w1i_ref[...], preferred_element_type=jnp.float32)
         + b1_ref[...])
    h = jnp.maximum(h, 0.0)
    h = jnp.dot(h, w2_ref[...], preferred_element_type=jnp.float32) + b2_ref[...]
    h = jnp.maximum(h, 0.0)
    out_ref[...] = jnp.sum(h * w3_ref[...], axis=1) + b3_ref[0]


def _mlp(ue, ie, w1u, w1i, b1r, w2, b2r, w3r, b3):
    grid = B // MLP_BLOCK
    rep2 = lambda shape: pl.BlockSpec(shape, lambda i: (0, 0))
    blk = pl.BlockSpec((MLP_BLOCK, LANES), lambda i: (i, 0))
    return pl.pallas_call(
        _mlp_body,
        grid=(grid,),
        in_specs=[
            blk, blk,
            rep2((EMB, HID)),
            rep2((EMB, HID)),
            rep2((1, HID)),
            rep2((HID, HID // 2)),
            rep2((1, HID // 2)),
            rep2((1, HID // 2)),
            pl.BlockSpec(memory_space=pltpu.SMEM),
        ],
        out_specs=pl.BlockSpec((MLP_BLOCK,), lambda i: (i,)),
        out_shape=jax.ShapeDtypeStruct((B,), jnp.float32),
    )(ue, ie, w1u, w1i, b1r, w2, b2r, w3r, b3)


def kernel(user_id, item_id, user_table, item_table, W1, b1, W2, b2, W3, b3):
    ue, ie = _sc_gather_fn()(user_id, item_id, user_table.T, item_table.T)
    w1u = W1[:, :EMB].T
    w1i = W1[:, EMB:].T
    return _mlp(ue, ie, w1u, w1i, b1[None, :], W2.T, b2[None, :], W3, b3)


# speedup vs baseline: 1.0385x; 1.0385x over previous
import functools

import jax
import jax.numpy as jnp
from jax import lax
from jax.experimental import pallas as pl
from jax.experimental.pallas import tpu as pltpu
from jax.experimental.pallas import tpu_sc as plsc

NUM_USERS = 1000000
NUM_ITEMS = 1000000
EMB = 32
HID = 64
B = 16384

NC = 2
NS = 16
NW = NC * NS
LANES = 128
TCOLS = (NUM_USERS + LANES - 1) // LANES
SPW = -(-TCOLS // NW)
WT = 4
WAVE = WT * LANES
NWAVES = -(-SPW // WT)
STRIPE = SPW * LANES
SLACK = 768
NGRP = SLACK // 16
WSLACK = 64
OUTROWS = B + 16


NBINS = 8
BINSZ = 256
BINSHIFT = 12


def _scan_idx(uids_v, iids_v, lo, hi, uhr, uhp, ihr, ihp):
    lanei = lax.iota(jnp.int32, 16)

    def body(v, offs):
        uo, io = offs
        pvec = lanei + v * 16
        urv = uids_v[pl.ds(v * 16, 16)]
        um = (urv >= lo) & (urv < hi)
        plsc.store_compressed(uhr.at[pl.ds(uo[0], 16)], urv, mask=um)
        plsc.store_compressed(uhp.at[pl.ds(uo[0], 16)], pvec, mask=um)
        irv = iids_v[pl.ds(v * 16, 16)]
        im = (irv >= lo) & (irv < hi)
        plsc.store_compressed(ihr.at[pl.ds(io[0], 16)], irv, mask=im)
        plsc.store_compressed(ihp.at[pl.ds(io[0], 16)], pvec, mask=im)
        return (uo + plsc.all_reduce_population_count(um),
                io + plsc.all_reduce_population_count(im))

    z = jnp.zeros((16,), jnp.int32)
    uo, io = lax.fori_loop(0, B // 16, body, (z, z))
    return uo[0], io[0]


def _partition(hr, hp, cnt, lo, br, bp):
    lanei = lax.iota(jnp.int32, 16)
    ngr = (cnt + 15) // 16
    cnts = []
    for b in range(NBINS):
        def body(v, off, b=b):
            rvec = hr[pl.ds(v * 16, 16)]
            pvec = hp[pl.ds(v * 16, 16)]
            m = (((rvec - lo) >> BINSHIFT) == b) & ((lanei + v * 16) < cnt)
            o = off[0]
            plsc.store_compressed(br.at[pl.ds(b * BINSZ + o, 16)], rvec,
                                  mask=m)
            plsc.store_compressed(bp.at[pl.ds(b * BINSZ + o, 16)], pvec,
                                  mask=m)
            return off + plsc.all_reduce_population_count(m)

        cnts.append(lax.fori_loop(0, ngr, body, jnp.zeros((16,), jnp.int32))[0])
    return cnts


def _gather_body(uid_hbm, iid_hbm, utabT_hbm, itabT_hbm, uout_hbm, iout_hbm,
                 uids_v, iids_v, uhr, uhp, ihr, ihp, ubr, ubp, ibr, ibp,
                 uwbr, uwbp, iwbr, iwbp, bcnt_v,
                 ubufs, ibufs, batch_v, bpos_v, fsems, ssem):
    c = lax.axis_index("c")
    s = lax.axis_index("s")
    wid = s * NC + c
    lo = wid * STRIPE
    hi = lo + STRIPE

    rows0 = lax.iota(jnp.int32, 16)
    rows1 = rows0 + 16
    lanei = rows0

    def wave_off(k):
        col = jnp.minimum(wid * SPW + k * WT, TCOLS - WT)
        return pl.multiple_of(col * LANES, LANES)

    def fetch(k, slot):
        off = wave_off(k)
        pltpu.async_copy(utabT_hbm.at[:, pl.ds(off, WAVE)], ubufs[slot],
                         fsems[2 * slot])
        pltpu.async_copy(itabT_hbm.at[:, pl.ds(off, WAVE)], ibufs[slot],
                         fsems[2 * slot + 1])

    def wait_fetch(slot):
        pltpu.make_async_copy(utabT_hbm.at[:, pl.ds(0, WAVE)], ubufs[slot],
                              fsems[2 * slot]).wait()
        pltpu.make_async_copy(itabT_hbm.at[:, pl.ds(0, WAVE)], ibufs[slot],
                              fsems[2 * slot + 1]).wait()

    def wait_scatter():
        pltpu.make_async_copy(batch_v, uout_hbm.at[pl.ds(0, 16)],
                              ssem).wait()

    fetch(0, 0)
    fetch(1, 1)

    pltpu.sync_copy(uid_hbm, uids_v)
    pltpu.sync_copy(iid_hbm, iids_v)
    ucnt, icnt = _scan_idx(uids_v, iids_v, lo, hi, uhr, uhp, ihr, ihp)

    sent = jnp.full((16,), -1, jnp.int32)

    def init_bins(v, carry):
        ubr[pl.ds(v * 16, 16)] = sent
        ibr[pl.ds(v * 16, 16)] = sent
        return carry

    lax.fori_loop(0, NBINS * BINSZ // 16, init_bins, 0)
    ucnts = _partition(uhr, uhp, ucnt, lo, ubr, ubp)
    icnts = _partition(ihr, ihp, icnt, lo, ibr, ibp)
    cvec = jnp.zeros((16,), jnp.int32)
    for b in range(NBINS):
        cvec = jnp.where(lanei == b, ucnts[b], cvec)
        cvec = jnp.where(lanei == NBINS + b, icnts[b], cvec)
    bcnt_v[pl.ds(0, 16)] = cvec

    def sub_partition(b):
        def init_wb(v, carry):
            uwbr[pl.ds(v * 16, 16)] = sent
            iwbr[pl.ds(v * 16, 16)] = sent
            return carry

        lax.fori_loop(0, NBINS * WSLACK // 16, init_wb, 0)
        ubn = (plsc.load_gather(bcnt_v, [jnp.full((16,), b, jnp.int32)])[0]
               + 15) // 16
        ibn = (plsc.load_gather(bcnt_v,
                                [jnp.full((16,), NBINS + b, jnp.int32)])[0]
               + 15) // 16
        bin0 = b * BINSZ
        for wv in range(NBINS):
            def ubody(v, off, wv=wv):
                rvec = ubr[pl.ds(bin0 + v * 16, 16)]
                pvec = ubp[pl.ds(bin0 + v * 16, 16)]
                m = ((((rvec - lo) >> 9) & 7) == wv) & (rvec >= 0)
                o = off[0]
                plsc.store_compressed(uwbr.at[pl.ds(wv * WSLACK + o, 16)],
                                      rvec, mask=m)
                plsc.store_compressed(uwbp.at[pl.ds(wv * WSLACK + o, 16)],
                                      pvec, mask=m)
                return off + plsc.all_reduce_population_count(m)

            lax.fori_loop(0, ubn, ubody, jnp.zeros((16,), jnp.int32))

            def ibody(v, off, wv=wv):
                rvec = ibr[pl.ds(bin0 + v * 16, 16)]
                pvec = ibp[pl.ds(bin0 + v * 16, 16)]
                m = ((((rvec - lo) >> 9) & 7) == wv) & (rvec >= 0)
                o = off[0]
                plsc.store_compressed(iwbr.at[pl.ds(wv * WSLACK + o, 16)],
                                      rvec, mask=m)
                plsc.store_compressed(iwbp.at[pl.ds(wv * WSLACK + o, 16)],
                                      pvec, mask=m)
                return off + plsc.all_reduce_population_count(m)

            lax.fori_loop(0, ibn, ibody, jnp.zeros((16,), jnp.int32))

    def extract_wave(k, buf, wbr, wbp, out_hbm, bcnt):
        fbase = wave_off(k)
        wb0 = (k & 7) * WSLACK

        def group(v, bcnt):
            rvec = wbr[pl.ds(wb0 + v * 16, 16)]
            pvec = wbp[pl.ds(wb0 + v * 16, 16)]
            m = rvec >= 0
            nhit = plsc.all_reduce_population_count(m)[0]

            def do_batch(bcnt):
                @pl.when(bcnt > 0)
                def _():
                    wait_scatter()
                pm = jnp.where(m, pvec, B + s)
                bpos_v[pl.ds(0, 16)] = pm
                for j in range(16):
                    @pl.when(rvec[j] >= 0)
                    def _():
                        colv = jnp.full((16,), rvec[j] - fbase, jnp.int32)
                        batch_v[j, pl.ds(0, 16)] = plsc.load_gather(
                            buf, [rows0, colv])
                        batch_v[j, pl.ds(16, 16)] = plsc.load_gather(
                            buf, [rows1, colv])
                pltpu.async_copy(batch_v, out_hbm.at[bpos_v], ssem)
                return bcnt + 1

            return lax.cond(nhit > 0, do_batch, lambda x: x, bcnt)

        return lax.fori_loop(0, WSLACK // 16, group, bcnt)

    def wave_pair(g, bcnt):
        for slot in range(2):
            k = g * 2 + slot

            @pl.when((k & 7) == 0)
            def _():
                sub_partition(k >> 3)

            wait_fetch(slot)
            bcnt = extract_wave(k, ubufs[slot], uwbr, uwbp, uout_hbm, bcnt)
            bcnt = extract_wave(k, ibufs[slot], iwbr, iwbp, iout_hbm, bcnt)
            nxt = k + 2

            @pl.when(nxt < NWAVES)
            def _():
                fetch(nxt, slot)
        return bcnt

    bcnt = lax.fori_loop(0, NWAVES // 2, wave_pair, 0)

    @pl.when(bcnt > 0)
    def _():
        wait_scatter()


def _gather_entry(uid_hbm, iid_hbm, utabT_hbm, itabT_hbm, uout, iout, *scr):
    (uids_v, iids_v, uhr, uhp, ihr, ihp, ubr, ubp, ibr, ibp,
     uwbr, uwbp, iwbr, iwbp, bcnt_v,
     ub0, ub1, ib0, ib1, batch_v, bpos_v, f0, f1, f2, f3, ssem) = scr
    _gather_body(uid_hbm, iid_hbm, utabT_hbm, itabT_hbm, uout, iout,
                 uids_v, iids_v, uhr, uhp, ihr, ihp, ubr, ubp, ibr, ibp,
                 uwbr, uwbp, iwbr, iwbp, bcnt_v,
                 [ub0, ub1], [ib0, ib1], batch_v, bpos_v,
                 [f0, f1, f2, f3], ssem)


@functools.cache
def _sc_gather_fn():
    wavebuf = pltpu.VMEM((EMB, WAVE), jnp.float32)
    hits = pltpu.VMEM((SLACK,), jnp.int32)
    whits = pltpu.VMEM((WSLACK,), jnp.int32)
    out_t = jax.ShapeDtypeStruct((OUTROWS, LANES), jnp.float32)
    return pl.kernel(
        _gather_entry,
        out_type=[out_t, out_t],
        mesh=plsc.VectorSubcoreMesh(core_axis_name="c", subcore_axis_name="s"),
        scratch_types=(
            [pltpu.VMEM((B,), jnp.int32)] * 2
            + [hits] * 4
            + [pltpu.VMEM((NBINS * BINSZ,), jnp.int32)] * 4
            + [pltpu.VMEM((NBINS * WSLACK,), jnp.int32)] * 4
            + [pltpu.VMEM((16,), jnp.int32)]
            + [wavebuf] * 4
            + [pltpu.VMEM((16, LANES), jnp.float32),
               pltpu.VMEM((16,), jnp.int32)]
            + [pltpu.SemaphoreType.DMA] * 5
        ),
        compiler_params=pltpu.CompilerParams(use_tc_tiling_on_sc=True,
                                             needs_layout_passes=False),
    )


MLP_BLOCK = 2048


def _mlp_body(ue_ref, ie_ref, w1u_ref, w1i_ref, b1_ref,
              w2_ref, b2_ref, w3_ref, b3_ref, out_ref):
    ue = ue_ref[...][:, :EMB]
    ie = ie_ref[...][:, :EMB]
    h = (jnp.dot(ue, w1u_ref[...], preferred_element_type=jnp.float32)
         + jnp.dot(ie, w1i_ref[...], preferred_element_type=jnp.float32)
         + b1_ref[...])
    h = jnp.maximum(h, 0.0)
    h = jnp.dot(h, w2_ref[...], preferred_element_type=jnp.float32) + b2_ref[...]
    h = jnp.maximum(h, 0.0)
    out_ref[...] = jnp.sum(h * w3_ref[...], axis=1) + b3_ref[0]


def _mlp(ue, ie, w1u, w1i, b1r, w2, b2r, w3r, b3):
    grid = B // MLP_BLOCK
    rep2 = lambda shape: pl.BlockSpec(shape, lambda i: (0, 0))
    blk = pl.BlockSpec((MLP_BLOCK, LANES), lambda i: (i, 0))
    return pl.pallas_call(
        _mlp_body,
        grid=(grid,),
        in_specs=[
            blk, blk,
            rep2((EMB, HID)),
            rep2((EMB, HID)),
            rep2((1, HID)),
            rep2((HID, HID // 2)),
            rep2((1, HID // 2)),
            rep2((1, HID // 2)),
            pl.BlockSpec(memory_space=pltpu.SMEM),
        ],
        out_specs=pl.BlockSpec((MLP_BLOCK,), lambda i: (i,)),
        out_shape=jax.ShapeDtypeStruct((B,), jnp.float32),
    )(ue, ie, w1u, w1i, b1r, w2, b2r, w3r, b3)


def kernel(user_id, item_id, user_table, item_table, W1, b1, W2, b2, W3, b3):
    ue, ie = _sc_gather_fn()(user_id, item_id, user_table.T, item_table.T)
    w1u = W1[:, :EMB].T
    w1i = W1[:, EMB:].T
    return _mlp(ue, ie, w1u, w1i, b1[None, :], W2.T, b2[None, :], W3, b3)


# speedup vs baseline: 1.4106x; 1.3584x over previous
import functools

import jax
import jax.numpy as jnp
from jax import lax
from jax.experimental import pallas as pl
from jax.experimental.pallas import tpu as pltpu
from jax.experimental.pallas import tpu_sc as plsc

NUM_USERS = 1000000
NUM_ITEMS = 1000000
EMB = 32
HID = 64
B = 16384

NC = 2
NS = 16
NW = NC * NS
BPW = B // NW
LANES = 128
RING = 8


GRP = 16


def _gather_body(uid_hbm, iid_hbm, utabT_hbm, itabT_hbm, uoutT_hbm, ioutT_hbm,
                 uidx_v, iidx_v, ubufs, ibufs, uout_v, iout_v, usems, isems):
    wid = lax.axis_index("s") * NC + lax.axis_index("c")
    base = wid * BPW
    pltpu.sync_copy(uid_hbm.at[pl.ds(base, BPW)], uidx_v)
    pltpu.sync_copy(iid_hbm.at[pl.ds(base, BPW)], iidx_v)

    rows0 = lax.iota(jnp.int32, 16)
    rows1 = rows0 + 16

    def fetch(ur, ir, slot):
        urt = pl.multiple_of((ur // LANES) * LANES, LANES)
        irt = pl.multiple_of((ir // LANES) * LANES, LANES)
        pltpu.async_copy(utabT_hbm.at[:, pl.ds(urt, LANES)], ubufs[slot],
                         usems[slot])
        pltpu.async_copy(itabT_hbm.at[:, pl.ds(irt, LANES)], ibufs[slot],
                         isems[slot])

    def wait(slot):
        pltpu.make_async_copy(utabT_hbm.at[:, pl.ds(0, LANES)], ubufs[slot],
                              usems[slot]).wait()
        pltpu.make_async_copy(itabT_hbm.at[:, pl.ds(0, LANES)], ibufs[slot],
                              isems[slot]).wait()

    def extract(i, ur, ir, slot):
        cols = jnp.full((16,), i, jnp.int32)
        url = jnp.full((16,), ur % LANES, jnp.int32)
        irl = jnp.full((16,), ir % LANES, jnp.int32)
        plsc.store_scatter(uout_v, [rows0, cols],
                           plsc.load_gather(ubufs[slot], [rows0, url]))
        plsc.store_scatter(uout_v, [rows1, cols],
                           plsc.load_gather(ubufs[slot], [rows1, url]))
        plsc.store_scatter(iout_v, [rows0, cols],
                           plsc.load_gather(ibufs[slot], [rows0, irl]))
        plsc.store_scatter(iout_v, [rows1, cols],
                           plsc.load_gather(ibufs[slot], [rows1, irl]))

    uv0 = uidx_v[pl.ds(0, GRP)]
    iv0 = iidx_v[pl.ds(0, GRP)]
    for p in range(RING):
        fetch(uv0[p], iv0[p], p)

    def body(g, carry):
        i0 = g * GRP
        uv = uidx_v[pl.ds(i0, GRP)]
        iv = iidx_v[pl.ds(i0, GRP)]
        nxt0 = jnp.minimum(i0 + GRP, BPW - GRP)
        uvn = uidx_v[pl.ds(nxt0, GRP)]
        ivn = iidx_v[pl.ds(nxt0, GRP)]
        for s in range(GRP):
            slot = s % RING
            wait(slot)
            extract(i0 + s, uv[s], iv[s], slot)
            if s < GRP - RING:
                fetch(uv[s + RING], iv[s + RING], slot)
            else:
                @pl.when(g < BPW // GRP - 1)
                def _():
                    fetch(uvn[s - (GRP - RING)], ivn[s - (GRP - RING)], slot)
        return carry

    lax.fori_loop(0, BPW // GRP, body, 0)
    pltpu.sync_copy(uout_v, uoutT_hbm.at[:, pl.ds(base, BPW)])
    pltpu.sync_copy(iout_v, ioutT_hbm.at[:, pl.ds(base, BPW)])


def _gather_entry(uid_hbm, iid_hbm, utabT_hbm, itabT_hbm, uoutT_hbm,
                  ioutT_hbm, *scr):
    uidx_v, iidx_v = scr[0], scr[1]
    ubufs = list(scr[2:2 + RING])
    ibufs = list(scr[2 + RING:2 + 2 * RING])
    uout_v, iout_v = scr[2 + 2 * RING], scr[3 + 2 * RING]
    usems = list(scr[4 + 2 * RING:4 + 3 * RING])
    isems = list(scr[4 + 3 * RING:4 + 4 * RING])
    _gather_body(uid_hbm, iid_hbm, utabT_hbm, itabT_hbm, uoutT_hbm, ioutT_hbm,
                 uidx_v, iidx_v, ubufs, ibufs, uout_v, iout_v, usems, isems)


@functools.cache
def _sc_gather_fn():
    buf = pltpu.VMEM((EMB, LANES), jnp.float32)
    return pl.kernel(
        _gather_entry,
        out_type=[
            jax.ShapeDtypeStruct((EMB, B), jnp.float32),
            jax.ShapeDtypeStruct((EMB, B), jnp.float32),
        ],
        mesh=plsc.VectorSubcoreMesh(core_axis_name="c", subcore_axis_name="s"),
        scratch_types=(
            [pltpu.VMEM((BPW,), jnp.int32)] * 2
            + [buf] * (2 * RING)
            + [pltpu.VMEM((EMB, BPW), jnp.float32)] * 2
            + [pltpu.SemaphoreType.DMA] * (2 * RING)
        ),
        compiler_params=pltpu.CompilerParams(use_tc_tiling_on_sc=True,
                                             needs_layout_passes=False),
    )


MLP_BLOCK = 2048


def _mlp_body(ue_ref, ie_ref, w1u_ref, w1i_ref, b1_ref, w2_ref, b2_ref,
              w3_ref, b3_ref, out_ref):
    h = (jnp.dot(w1u_ref[...], ue_ref[...], preferred_element_type=jnp.float32)
         + jnp.dot(w1i_ref[...], ie_ref[...], preferred_element_type=jnp.float32)
         + b1_ref[...])
    h = jnp.maximum(h, 0.0)
    h = jnp.dot(w2_ref[...], h, preferred_element_type=jnp.float32) + b2_ref[...]
    h = jnp.maximum(h, 0.0)
    out_ref[...] = jnp.sum(h * w3_ref[...], axis=0) + b3_ref[0]


def _mlp(ueT, ieT, w1u, w1i, b1c, w2, b2c, w3c, b3):
    grid = B // MLP_BLOCK
    rep2 = lambda shape: pl.BlockSpec(shape, lambda i: (0, 0))
    return pl.pallas_call(
        _mlp_body,
        grid=(grid,),
        in_specs=[
            pl.BlockSpec((EMB, MLP_BLOCK), lambda i: (0, i)),
            pl.BlockSpec((EMB, MLP_BLOCK), lambda i: (0, i)),
            rep2((HID, EMB)),
            rep2((HID, EMB)),
            rep2((HID, 1)),
            rep2((HID // 2, HID)),
            rep2((HID // 2, 1)),
            rep2((HID // 2, 1)),
            pl.BlockSpec(memory_space=pltpu.SMEM),
        ],
        out_specs=pl.BlockSpec((MLP_BLOCK,), lambda i: (i,)),
        out_shape=jax.ShapeDtypeStruct((B,), jnp.float32),
    )(ueT, ieT, w1u, w1i, b1c, w2, b2c, w3c, b3)


def kernel(user_id, item_id, user_table, item_table, W1, b1, W2, b2, W3, b3):
    ueT, ieT = _sc_gather_fn()(user_id, item_id, user_table.T, item_table.T)
    w1u = W1[:, :EMB]
    w1i = W1[:, EMB:]
    return _mlp(ueT, ieT, w1u, w1i, b1[:, None], W2, b2[:, None],
                W3[0][:, None], b3)


# speedup vs baseline: 1.4247x; 1.0099x over previous
import functools

import jax
import jax.numpy as jnp
from jax import lax
from jax.experimental import pallas as pl
from jax.experimental.pallas import tpu as pltpu
from jax.experimental.pallas import tpu_sc as plsc

NUM_USERS = 1000000
NUM_ITEMS = 1000000
EMB = 32
HID = 64
B = 16384

NC = 2
NS = 16
NW = NC * NS
BPW = B // NW
LANES = 128
RING = 8


GRP = 16


def _gather_body(uid_hbm, iid_hbm, utabT_hbm, itabT_hbm, uoutT_hbm, ioutT_hbm,
                 uidx_v, iidx_v, ubufs, ibufs, uout_v, iout_v, usems, isems):
    wid = lax.axis_index("s") * NC + lax.axis_index("c")
    base = wid * BPW
    pltpu.sync_copy(uid_hbm.at[pl.ds(base, BPW)], uidx_v)
    pltpu.sync_copy(iid_hbm.at[pl.ds(base, BPW)], iidx_v)

    rows0 = lax.iota(jnp.int32, 16)
    rows1 = rows0 + 16

    def fetch(ur, ir, slot):
        urt = pl.multiple_of((ur // LANES) * LANES, LANES)
        irt = pl.multiple_of((ir // LANES) * LANES, LANES)
        pltpu.async_copy(utabT_hbm.at[:, pl.ds(urt, LANES)], ubufs[slot],
                         usems[slot])
        pltpu.async_copy(itabT_hbm.at[:, pl.ds(irt, LANES)], ibufs[slot],
                         isems[slot])

    def wait(slot):
        pltpu.make_async_copy(utabT_hbm.at[:, pl.ds(0, LANES)], ubufs[slot],
                              usems[slot]).wait()
        pltpu.make_async_copy(itabT_hbm.at[:, pl.ds(0, LANES)], ibufs[slot],
                              isems[slot]).wait()

    def extract(i, ur, ir, slot):
        cols = jnp.full((16,), i, jnp.int32)
        url = jnp.full((16,), ur % LANES, jnp.int32)
        irl = jnp.full((16,), ir % LANES, jnp.int32)
        plsc.store_scatter(uout_v, [rows0, cols],
                           plsc.load_gather(ubufs[slot], [rows0, url]))
        plsc.store_scatter(uout_v, [rows1, cols],
                           plsc.load_gather(ubufs[slot], [rows1, url]))
        plsc.store_scatter(iout_v, [rows0, cols],
                           plsc.load_gather(ibufs[slot], [rows0, irl]))
        plsc.store_scatter(iout_v, [rows1, cols],
                           plsc.load_gather(ibufs[slot], [rows1, irl]))

    uv0 = uidx_v[pl.ds(0, GRP)]
    iv0 = iidx_v[pl.ds(0, GRP)]
    for p in range(RING):
        fetch(uv0[p], iv0[p], p)

    def body(g, carry):
        i0 = g * GRP
        uv = uidx_v[pl.ds(i0, GRP)]
        iv = iidx_v[pl.ds(i0, GRP)]
        nxt0 = jnp.minimum(i0 + GRP, BPW - GRP)
        uvn = uidx_v[pl.ds(nxt0, GRP)]
        ivn = iidx_v[pl.ds(nxt0, GRP)]
        for s in range(GRP):
            slot = s % RING
            wait(slot)
            extract(i0 + s, uv[s], iv[s], slot)
            if s < GRP - RING:
                fetch(uv[s + RING], iv[s + RING], slot)
            else:
                @pl.when(g < BPW // GRP - 1)
                def _():
                    fetch(uvn[s - (GRP - RING)], ivn[s - (GRP - RING)], slot)
        return carry

    lax.fori_loop(0, BPW // GRP, body, 0)
    pltpu.sync_copy(uout_v, uoutT_hbm.at[:, pl.ds(base, BPW)])
    pltpu.sync_copy(iout_v, ioutT_hbm.at[:, pl.ds(base, BPW)])


def _gather_entry(uid_hbm, iid_hbm, utabT_hbm, itabT_hbm, uoutT_hbm,
                  ioutT_hbm, *scr):
    uidx_v, iidx_v = scr[0], scr[1]
    ubufs = list(scr[2:2 + RING])
    ibufs = list(scr[2 + RING:2 + 2 * RING])
    uout_v, iout_v = scr[2 + 2 * RING], scr[3 + 2 * RING]
    usems = list(scr[4 + 2 * RING:4 + 3 * RING])
    isems = list(scr[4 + 3 * RING:4 + 4 * RING])
    _gather_body(uid_hbm, iid_hbm, utabT_hbm, itabT_hbm, uoutT_hbm, ioutT_hbm,
                 uidx_v, iidx_v, ubufs, ibufs, uout_v, iout_v, usems, isems)


@functools.cache
def _sc_gather_fn():
    buf = pltpu.VMEM((EMB, LANES), jnp.float32)
    return pl.kernel(
        _gather_entry,
        out_type=[
            jax.ShapeDtypeStruct((EMB, B), jnp.float32),
            jax.ShapeDtypeStruct((EMB, B), jnp.float32),
        ],
        mesh=plsc.VectorSubcoreMesh(core_axis_name="c", subcore_axis_name="s"),
        scratch_types=(
            [pltpu.VMEM((BPW,), jnp.int32)] * 2
            + [buf] * (2 * RING)
            + [pltpu.VMEM((EMB, BPW), jnp.float32)] * 2
            + [pltpu.SemaphoreType.DMA] * (2 * RING)
        ),
        compiler_params=pltpu.CompilerParams(use_tc_tiling_on_sc=True,
                                             needs_layout_passes=False),
    )


MLP_BLOCK = 4096


def _mlp_body(ue_ref, ie_ref, w1u_ref, w1i_ref, b1_ref, w2_ref, b2_ref,
              w3_ref, b3_ref, out_ref):
    h = (jnp.dot(w1u_ref[...], ue_ref[...], preferred_element_type=jnp.float32)
         + jnp.dot(w1i_ref[...], ie_ref[...], preferred_element_type=jnp.float32)
         + b1_ref[...])
    h = jnp.maximum(h, 0.0)
    h = jnp.dot(w2_ref[...], h, preferred_element_type=jnp.float32) + b2_ref[...]
    h = jnp.maximum(h, 0.0)
    out_ref[...] = jnp.sum(h * w3_ref[...], axis=0) + b3_ref[0]


def _mlp(ueT, ieT, w1u, w1i, b1c, w2, b2c, w3c, b3):
    grid = B // MLP_BLOCK
    rep2 = lambda shape: pl.BlockSpec(shape, lambda i: (0, 0))
    return pl.pallas_call(
        _mlp_body,
        grid=(grid,),
        in_specs=[
            pl.BlockSpec((EMB, MLP_BLOCK), lambda i: (0, i)),
            pl.BlockSpec((EMB, MLP_BLOCK), lambda i: (0, i)),
            rep2((HID, EMB)),
            rep2((HID, EMB)),
            rep2((HID, 1)),
            rep2((HID // 2, HID)),
            rep2((HID // 2, 1)),
            rep2((HID // 2, 1)),
            pl.BlockSpec(memory_space=pltpu.SMEM),
        ],
        out_specs=pl.BlockSpec((MLP_BLOCK,), lambda i: (i,)),
        out_shape=jax.ShapeDtypeStruct((B,), jnp.float32),
    )(ueT, ieT, w1u, w1i, b1c, w2, b2c, w3c, b3)


def kernel(user_id, item_id, user_table, item_table, W1, b1, W2, b2, W3, b3):
    ueT, ieT = _sc_gather_fn()(user_id, item_id, user_table.T, item_table.T)
    w1u = W1[:, :EMB]
    w1i = W1[:, EMB:]
    return _mlp(ueT, ieT, w1u, w1i, b1[:, None], W2, b2[:, None],
                W3[0][:, None], b3)


# speedup vs baseline: 1.4373x; 1.0089x over previous
import functools

import jax
import jax.numpy as jnp
from jax import lax
from jax.experimental import pallas as pl
from jax.experimental.pallas import tpu as pltpu
from jax.experimental.pallas import tpu_sc as plsc

NUM_USERS = 1000000
NUM_ITEMS = 1000000
EMB = 32
HID = 64
B = 16384

NC = 2
NS = 16
NW = NC * NS
BPW = B // NW
LANES = 128
RING = 8


GRP = 16


def _gather_body(uid_hbm, iid_hbm, utabT_hbm, itabT_hbm, uoutT_hbm, ioutT_hbm,
                 uidx_v, iidx_v, ubufs, ibufs, uout_v, iout_v, usems, isems):
    wid = lax.axis_index("s") * NC + lax.axis_index("c")
    base = wid * BPW
    pltpu.sync_copy(uid_hbm.at[pl.ds(base, BPW)], uidx_v)
    pltpu.sync_copy(iid_hbm.at[pl.ds(base, BPW)], iidx_v)

    rows0 = lax.iota(jnp.int32, 16)
    rows1 = rows0 + 16

    def fetch(ur, ir, slot):
        urt = pl.multiple_of((ur // LANES) * LANES, LANES)
        irt = pl.multiple_of((ir // LANES) * LANES, LANES)
        pltpu.async_copy(utabT_hbm.at[:, pl.ds(urt, LANES)], ubufs[slot],
                         usems[slot])
        pltpu.async_copy(itabT_hbm.at[:, pl.ds(irt, LANES)], ibufs[slot],
                         isems[slot])

    def wait(slot):
        pltpu.make_async_copy(utabT_hbm.at[:, pl.ds(0, LANES)], ubufs[slot],
                              usems[slot]).wait()
        pltpu.make_async_copy(itabT_hbm.at[:, pl.ds(0, LANES)], ibufs[slot],
                              isems[slot]).wait()

    def extract(i, ur, ir, slot):
        cols = jnp.full((16,), i, jnp.int32)
        url = jnp.full((16,), ur % LANES, jnp.int32)
        irl = jnp.full((16,), ir % LANES, jnp.int32)
        plsc.store_scatter(uout_v, [rows0, cols],
                           plsc.load_gather(ubufs[slot], [rows0, url]))
        plsc.store_scatter(uout_v, [rows1, cols],
                           plsc.load_gather(ubufs[slot], [rows1, url]))
        plsc.store_scatter(iout_v, [rows0, cols],
                           plsc.load_gather(ibufs[slot], [rows0, irl]))
        plsc.store_scatter(iout_v, [rows1, cols],
                           plsc.load_gather(ibufs[slot], [rows1, irl]))

    uv0 = uidx_v[pl.ds(0, GRP)]
    iv0 = iidx_v[pl.ds(0, GRP)]
    for p in range(RING):
        fetch(uv0[p], iv0[p], p)

    def body(g, carry):
        i0 = g * GRP
        uv = uidx_v[pl.ds(i0, GRP)]
        iv = iidx_v[pl.ds(i0, GRP)]
        nxt0 = jnp.minimum(i0 + GRP, BPW - GRP)
        uvn = uidx_v[pl.ds(nxt0, GRP)]
        ivn = iidx_v[pl.ds(nxt0, GRP)]
        for s in range(GRP):
            slot = s % RING
            wait(slot)
            extract(i0 + s, uv[s], iv[s], slot)
            if s < GRP - RING:
                fetch(uv[s + RING], iv[s + RING], slot)
            else:
                @pl.when(g < BPW // GRP - 1)
                def _():
                    fetch(uvn[s - (GRP - RING)], ivn[s - (GRP - RING)], slot)
        return carry

    lax.fori_loop(0, BPW // GRP, body, 0)
    pltpu.sync_copy(uout_v, uoutT_hbm.at[:, pl.ds(base, BPW)])
    pltpu.sync_copy(iout_v, ioutT_hbm.at[:, pl.ds(base, BPW)])


def _gather_entry(uid_hbm, iid_hbm, utabT_hbm, itabT_hbm, uoutT_hbm,
                  ioutT_hbm, *scr):
    uidx_v, iidx_v = scr[0], scr[1]
    ubufs = list(scr[2:2 + RING])
    ibufs = list(scr[2 + RING:2 + 2 * RING])
    uout_v, iout_v = scr[2 + 2 * RING], scr[3 + 2 * RING]
    usems = list(scr[4 + 2 * RING:4 + 3 * RING])
    isems = list(scr[4 + 3 * RING:4 + 4 * RING])
    _gather_body(uid_hbm, iid_hbm, utabT_hbm, itabT_hbm, uoutT_hbm, ioutT_hbm,
                 uidx_v, iidx_v, ubufs, ibufs, uout_v, iout_v, usems, isems)


@functools.cache
def _sc_gather_fn():
    buf = pltpu.VMEM((EMB, LANES), jnp.float32)
    return pl.kernel(
        _gather_entry,
        out_type=[
            jax.ShapeDtypeStruct((EMB, B), jnp.float32),
            jax.ShapeDtypeStruct((EMB, B), jnp.float32),
        ],
        mesh=plsc.VectorSubcoreMesh(core_axis_name="c", subcore_axis_name="s"),
        scratch_types=(
            [pltpu.VMEM((BPW,), jnp.int32)] * 2
            + [buf] * (2 * RING)
            + [pltpu.VMEM((EMB, BPW), jnp.float32)] * 2
            + [pltpu.SemaphoreType.DMA] * (2 * RING)
        ),
        compiler_params=pltpu.CompilerParams(use_tc_tiling_on_sc=True,
                                             needs_layout_passes=False),
    )


MLP_BLOCK = 8192


def _mlp_body(ue_ref, ie_ref, w1u_ref, w1i_ref, b1_ref, w2_ref, b2_ref,
              w3_ref, b3_ref, out_ref):
    h = (jnp.dot(w1u_ref[...], ue_ref[...], preferred_element_type=jnp.float32)
         + jnp.dot(w1i_ref[...], ie_ref[...], preferred_element_type=jnp.float32)
         + b1_ref[...])
    h = jnp.maximum(h, 0.0)
    h = jnp.dot(w2_ref[...], h, preferred_element_type=jnp.float32) + b2_ref[...]
    h = jnp.maximum(h, 0.0)
    out_ref[...] = jnp.sum(h * w3_ref[...], axis=0) + b3_ref[0]


def _mlp(ueT, ieT, w1u, w1i, b1c, w2, b2c, w3c, b3):
    grid = B // MLP_BLOCK
    rep2 = lambda shape: pl.BlockSpec(shape, lambda i: (0, 0))
    return pl.pallas_call(
        _mlp_body,
        grid=(grid,),
        in_specs=[
            pl.BlockSpec((EMB, MLP_BLOCK), lambda i: (0, i)),
            pl.BlockSpec((EMB, MLP_BLOCK), lambda i: (0, i)),
            rep2((HID, EMB)),
            rep2((HID, EMB)),
            rep2((HID, 1)),
            rep2((HID // 2, HID)),
            rep2((HID // 2, 1)),
            rep2((HID // 2, 1)),
            pl.BlockSpec(memory_space=pltpu.SMEM),
        ],
        out_specs=pl.BlockSpec((MLP_BLOCK,), lambda i: (i,)),
        out_shape=jax.ShapeDtypeStruct((B,), jnp.float32),
    )(ueT, ieT, w1u, w1i, b1c, w2, b2c, w3c, b3)


def kernel(user_id, item_id, user_table, item_table, W1, b1, W2, b2, W3, b3):
    ueT, ieT = _sc_gather_fn()(user_id, item_id, user_table.T, item_table.T)
    w1u = W1[:, :EMB]
    w1i = W1[:, EMB:]
    return _mlp(ueT, ieT, w1u, w1i, b1[:, None], W2, b2[:, None],
                W3[0][:, None], b3)
